# baseline (speedup 1.0000x reference)
import jax
import jax.numpy as jnp
from jax.experimental import pallas as pl
from jax.experimental.pallas import tpu as pltpu


def kernel(feature, adj, shuf, sparse, msk, samp_bias1, samp_bias2,
           W_gcn, W_disc, b_disc, W_discAll, b_discAll, H):
    G, _, N, FT = feature.shape
    hid = W_gcn.shape[-1]
    f = feature.reshape(G, N, FT)
    s = shuf.reshape(G, N, FT)
    a = adj.reshape(G, N, N)
    bm = 512
    ni = N // bm
    av = adj.reshape(G, 2 * ni, bm // 2, N)

    def fused(f_ref, sh_ref, a_ref, ab_ref, w_ref, wd_ref, wda_ref, bd_ref, bda_ref,
              s1_ref, s2_ref, hr_ref, l0_ref, l1_ref, l2_ref, reg_ref,
              xw_s, h_s, ms0_s, ms1_s, rg_s):
        g = pl.program_id(0)
        i = pl.program_id(1)

        @pl.when(i == 0)
        def _():
            w = w_ref[0]
            p1 = jnp.dot(f_ref[0], w, preferred_element_type=jnp.float32)
            p2 = jnp.dot(sh_ref[0], w, preferred_element_type=jnp.float32)
            xw_s[...] = jnp.concatenate([p1, p2], axis=-1)

        hblk_t = jnp.maximum(
            jnp.dot(a_ref[0, 0], xw_s[...],
                    preferred_element_type=jnp.float32), 0.0)
        hblk_b = jnp.maximum(
            jnp.dot(ab_ref[0, 0], xw_s[...],
                    preferred_element_type=jnp.float32), 0.0)
        h_s[pl.ds(g * N + i * bm, bm // 2), :] = hblk_t
        h_s[pl.ds(g * N + i * bm + bm // 2, bm // 2), :] = hblk_b
        hblk = jnp.concatenate([hblk_t, hblk_b], axis=0)

        colsum = jnp.sum(hblk, axis=0, keepdims=True)

        @pl.when(jnp.logical_and(g == 0, i == 0))
        def _():
            ms0_s[...] = colsum

        @pl.when(jnp.logical_and(g == 0, i != 0))
        def _():
            ms0_s[...] += colsum

        @pl.when(jnp.logical_and(g == 1, i == 0))
        def _():
            ms1_s[...] = colsum

        @pl.when(jnp.logical_and(g == 1, i != 0))
        def _():
            ms1_s[...] += colsum

        @pl.when(g == 1)
        def _():
            h1a = 0.5 * (h_s[pl.ds(i * bm, bm), 0:hid] + hblk[:, 0:hid])
            h2a = 0.5 * (h_s[pl.ds(i * bm, bm), hid:] + hblk[:, hid:])
            hrb = hr_ref[0]
            d1 = hrb - h1a
            d2 = hrb - h2a
            rpart = jnp.sum(d1 * d1 - d2 * d2, axis=0, keepdims=True)

            @pl.when(i == 0)
            def _():
                rg_s[...] = rpart

            @pl.when(i != 0)
            def _():
                rg_s[...] += rpart

        @pl.when(jnp.logical_and(g == G - 1, i == ni - 1))
        def _():
            s1 = s1_ref[...]
            s2 = s2_ref[...]
            wd = wd_ref[...]
            wda = wda_ref[...]
            bd = bd_ref[...]
            bda = bda_ref[...]

            inv_n = 1.0 / N
            c0 = jax.nn.sigmoid(ms0_s[0:1, 0:hid] * inv_n)
            c1 = jax.nn.sigmoid(ms1_s[0:1, 0:hid] * inv_n)
            ca = jax.nn.sigmoid((ms0_s[0:1, 0:hid] + ms1_s[0:1, 0:hid])
                                * (0.5 * inv_n))
            wc0 = jnp.dot(wd, c0.T, preferred_element_type=jnp.float32)
            wc1 = jnp.dot(wd, c1.T, preferred_element_type=jnp.float32)
            wca = jnp.dot(wda, ca.T, preferred_element_type=jnp.float32)

            z = jnp.zeros_like(wc0)
            b0 = jnp.concatenate([
                jnp.concatenate([wc0, z, wca, z], axis=1),
                jnp.concatenate([z, wc0, z, wca], axis=1)], axis=0)
            b1 = jnp.concatenate([
                jnp.concatenate([wc1, z, wca, z], axis=1),
                jnp.concatenate([z, wc1, z, wca], axis=1)], axis=0)
            o0 = jnp.dot(h_s[0:N, :], b0, preferred_element_type=jnp.float32)
            o1 = jnp.dot(h_s[N:, :], b1, preferred_element_type=jnp.float32)
            t = jnp.concatenate([o0, o1], axis=1).T

            l0_ref[:, 0:N] = t[0:1] + bd + s1
            l0_ref[:, N:] = t[1:2] + bd + s2
            l1_ref[:, 0:N] = t[4:5] + bd + s1
            l1_ref[:, N:] = t[5:6] + bd + s2
            l2_ref[:, 0:N] = 0.5 * (t[2:3] + t[6:7]) + bda + s1
            l2_ref[:, N:] = 0.5 * (t[3:4] + t[7:8]) + bda + s2
            reg_ref[...] = jnp.sum(rg_s[0:1, 0:hid], axis=1, keepdims=True)

    l0, l1, l2, reg = pl.pallas_call(
        fused,
        grid=(G, ni),
        in_specs=[
            pl.BlockSpec((1, N, FT), lambda g, i: (g, 0, 0)),
            pl.BlockSpec((1, N, FT), lambda g, i: (g, 0, 0)),
            pl.BlockSpec((1, 1, bm // 2, N),
                         lambda g, i: (g, 2 * i, 0, 0)),
            pl.BlockSpec((1, 1, bm // 2, N),
                         lambda g, i: (g, 2 * i + 1, 0, 0)),
            pl.BlockSpec((1, FT, hid), lambda g, i: (g, 0, 0)),
            pl.BlockSpec((hid, hid), lambda g, i: (0, 0)),
            pl.BlockSpec((hid, hid), lambda g, i: (0, 0)),
            pl.BlockSpec((1, 1), lambda g, i: (0, 0)),
            pl.BlockSpec((1, 1), lambda g, i: (0, 0)),
            pl.BlockSpec((1, N), lambda g, i: (0, 0)),
            pl.BlockSpec((1, N), lambda g, i: (0, 0)),
            pl.BlockSpec((1, bm, hid), lambda g, i: (0, i * g, 0)),
        ],
        out_specs=[
            pl.BlockSpec((1, 2 * N), lambda g, i: (0, 0)),
            pl.BlockSpec((1, 2 * N), lambda g, i: (0, 0)),
            pl.BlockSpec((1, 2 * N), lambda g, i: (0, 0)),
            pl.BlockSpec((1, 1), lambda g, i: (0, 0)),
        ],
        out_shape=[
            jax.ShapeDtypeStruct((1, 2 * N), jnp.float32),
            jax.ShapeDtypeStruct((1, 2 * N), jnp.float32),
            jax.ShapeDtypeStruct((1, 2 * N), jnp.float32),
            jax.ShapeDtypeStruct((1, 1), jnp.float32),
        ],
        scratch_shapes=[
            pltpu.VMEM((N, 2 * hid), jnp.float32),
            pltpu.VMEM((G * N, 2 * hid), jnp.float32),
            pltpu.VMEM((1, 2 * hid), jnp.float32),
            pltpu.VMEM((1, 2 * hid), jnp.float32),
            pltpu.VMEM((1, hid), jnp.float32),
        ],
    )(f, s, av, av, W_gcn, W_disc, W_discAll,
      b_disc.reshape(1, 1), b_discAll.reshape(1, 1),
      samp_bias1, samp_bias2, H)

    return (l0, l1, l2, reg.reshape(()))


# speedup vs baseline: 1.0142x; 1.0142x over previous
import jax
import jax.numpy as jnp
from jax.experimental import pallas as pl
from jax.experimental.pallas import tpu as pltpu


def kernel(feature, adj, shuf, sparse, msk, samp_bias1, samp_bias2,
           W_gcn, W_disc, b_disc, W_discAll, b_discAll, H):
    G, _, N, FT = feature.shape
    hid = W_gcn.shape[-1]
    f = feature.reshape(G, N, FT)
    s = shuf.reshape(G, N, FT)
    a = adj.reshape(G, N, N)
    bm = 512
    ni = N // bm

    def fused(f_ref, sh_ref, a_ref, w_ref, wd_ref, wda_ref, bd_ref, bda_ref,
              s1_ref, s2_ref, hr_ref, l0_ref, l1_ref, l2_ref, reg_ref,
              xw_s, h_s, ms0_s, ms1_s, rg_s):
        g = pl.program_id(0)
        i = pl.program_id(1)

        @pl.when(i == 0)
        def _():
            w = w_ref[0]
            p1 = jnp.dot(f_ref[0], w, preferred_element_type=jnp.float32)
            p2 = jnp.dot(sh_ref[0], w, preferred_element_type=jnp.float32)
            xw_s[...] = jnp.concatenate([p1, p2], axis=-1)

        hblk = jnp.maximum(
            jnp.dot(a_ref[0], xw_s[...],
                    preferred_element_type=jnp.float32), 0.0)
        h_s[pl.ds(g * N + i * bm, bm), :] = hblk

        colsum = jnp.sum(hblk, axis=0, keepdims=True)

        @pl.when(jnp.logical_and(g == 0, i == 0))
        def _():
            ms0_s[...] = colsum

        @pl.when(jnp.logical_and(g == 0, i != 0))
        def _():
            ms0_s[...] += colsum

        @pl.when(jnp.logical_and(g == 1, i == 0))
        def _():
            ms1_s[...] = colsum

        @pl.when(jnp.logical_and(g == 1, i != 0))
        def _():
            ms1_s[...] += colsum

        @pl.when(g == 1)
        def _():
            h1a = 0.5 * (h_s[pl.ds(i * bm, bm), 0:hid] + hblk[:, 0:hid])
            h2a = 0.5 * (h_s[pl.ds(i * bm, bm), hid:] + hblk[:, hid:])
            hrb = hr_ref[0]
            d1 = hrb - h1a
            d2 = hrb - h2a
            rpart = jnp.sum(d1 * d1 - d2 * d2, axis=0, keepdims=True)

            @pl.when(i == 0)
            def _():
                rg_s[...] = rpart

            @pl.when(i != 0)
            def _():
                rg_s[...] += rpart

        @pl.when(jnp.logical_and(g == G - 1, i == ni - 1))
        def _():
            s1 = s1_ref[...]
            s2 = s2_ref[...]
            wd = wd_ref[...]
            wda = wda_ref[...]
            bd = bd_ref[...]
            bda = bda_ref[...]

            inv_n = 1.0 / N
            c0 = jax.nn.sigmoid(ms0_s[0:1, 0:hid] * inv_n)
            c1 = jax.nn.sigmoid(ms1_s[0:1, 0:hid] * inv_n)
            ca = jax.nn.sigmoid((ms0_s[0:1, 0:hid] + ms1_s[0:1, 0:hid])
                                * (0.5 * inv_n))
            wc0 = jnp.dot(wd, c0.T, preferred_element_type=jnp.float32)
            wc1 = jnp.dot(wd, c1.T, preferred_element_type=jnp.float32)
            wca = jnp.dot(wda, ca.T, preferred_element_type=jnp.float32)

            z = jnp.zeros_like(wc0)
            b0 = jnp.concatenate([
                jnp.concatenate([wc0, z, wca, z], axis=1),
                jnp.concatenate([z, wc0, z, wca], axis=1)], axis=0)
            b1 = jnp.concatenate([
                jnp.concatenate([wc1, z, wca, z], axis=1),
                jnp.concatenate([z, wc1, z, wca], axis=1)], axis=0)
            o0 = jnp.dot(h_s[0:N, :], b0, preferred_element_type=jnp.float32)
            o1 = jnp.dot(h_s[N:, :], b1, preferred_element_type=jnp.float32)
            t = jnp.concatenate([o0, o1], axis=1).T

            l0_ref[:, 0:N] = t[0:1] + bd + s1
            l0_ref[:, N:] = t[1:2] + bd + s2
            l1_ref[:, 0:N] = t[4:5] + bd + s1
            l1_ref[:, N:] = t[5:6] + bd + s2
            l2_ref[:, 0:N] = 0.5 * (t[2:3] + t[6:7]) + bda + s1
            l2_ref[:, N:] = 0.5 * (t[3:4] + t[7:8]) + bda + s2
            reg_ref[...] = jnp.sum(rg_s[0:1, 0:hid], axis=1, keepdims=True)

    l0, l1, l2, reg = pl.pallas_call(
        fused,
        grid=(G, ni),
        in_specs=[
            pl.BlockSpec((1, N, FT), lambda g, i: (g, 0, 0)),
            pl.BlockSpec((1, N, FT), lambda g, i: (g, 0, 0)),
            pl.BlockSpec((1, bm, N), lambda g, i: (g, i, 0)),
            pl.BlockSpec((1, FT, hid), lambda g, i: (g, 0, 0)),
            pl.BlockSpec((hid, hid), lambda g, i: (0, 0)),
            pl.BlockSpec((hid, hid), lambda g, i: (0, 0)),
            pl.BlockSpec((1, 1), lambda g, i: (0, 0)),
            pl.BlockSpec((1, 1), lambda g, i: (0, 0)),
            pl.BlockSpec((1, N), lambda g, i: (0, 0)),
            pl.BlockSpec((1, N), lambda g, i: (0, 0)),
            pl.BlockSpec((1, bm, hid), lambda g, i: (0, i * g, 0)),
        ],
        out_specs=[
            pl.BlockSpec((1, 2 * N), lambda g, i: (0, 0)),
            pl.BlockSpec((1, 2 * N), lambda g, i: (0, 0)),
            pl.BlockSpec((1, 2 * N), lambda g, i: (0, 0)),
            pl.BlockSpec((1, 1), lambda g, i: (0, 0)),
        ],
        out_shape=[
            jax.ShapeDtypeStruct((1, 2 * N), jnp.float32),
            jax.ShapeDtypeStruct((1, 2 * N), jnp.float32),
            jax.ShapeDtypeStruct((1, 2 * N), jnp.float32),
            jax.ShapeDtypeStruct((1, 1), jnp.float32),
        ],
        scratch_shapes=[
            pltpu.VMEM((N, 2 * hid), jnp.float32),
            pltpu.VMEM((G * N, 2 * hid), jnp.float32),
            pltpu.VMEM((1, 2 * hid), jnp.float32),
            pltpu.VMEM((1, 2 * hid), jnp.float32),
            pltpu.VMEM((1, hid), jnp.float32),
        ],
    )(f, s, a, W_gcn, W_disc, W_discAll,
      b_disc.reshape(1, 1), b_discAll.reshape(1, 1),
      samp_bias1, samp_bias2, H)

    return (l0, l1, l2, reg.reshape(()))
